# baseline (speedup 1.0000x reference)
import jax
import jax.numpy as jnp
from jax.experimental import pallas as pl
from jax.experimental.pallas import tpu as pltpu

_ROWS = 16384
_COLS = 256
_BLOCK_ROWS = 7936


def _copy_body(t_ref, q_ref, t_out, q_out):
    t_out[...] = t_ref[...]
    q_out[...] = q_ref[...]


def kernel(emb_targets, emb_queries):
    grid = (-(-_ROWS // _BLOCK_ROWS),)
    spec = pl.BlockSpec((_BLOCK_ROWS, _COLS), lambda i: (i, 0))
    out_t, out_q = pl.pallas_call(
        _copy_body,
        grid=grid,
        in_specs=[spec, spec],
        out_specs=[spec, spec],
        out_shape=[
            jax.ShapeDtypeStruct((_ROWS, _COLS), jnp.float32),
            jax.ShapeDtypeStruct((_ROWS, _COLS), jnp.float32),
        ],
        compiler_params=pltpu.CompilerParams(vmem_limit_bytes=100 * 1024 * 1024),
    )(emb_targets, emb_queries)
    return (out_t, out_q)


# speedup vs baseline: 1.0264x; 1.0264x over previous
import jax
import jax.numpy as jnp
from jax.experimental import pallas as pl
from jax.experimental.pallas import tpu as pltpu

_ROWS = 16384
_COLS = 256
_BLOCK_ROWS = 7424


def _copy_body(t_ref, q_ref, t_out, q_out):
    t_out[...] = t_ref[...]
    q_out[...] = q_ref[...]


def kernel(emb_targets, emb_queries):
    grid = (-(-_ROWS // _BLOCK_ROWS),)
    spec = pl.BlockSpec((_BLOCK_ROWS, _COLS), lambda i: (i, 0))
    out_t, out_q = pl.pallas_call(
        _copy_body,
        grid=grid,
        in_specs=[spec, spec],
        out_specs=[spec, spec],
        out_shape=[
            jax.ShapeDtypeStruct((_ROWS, _COLS), jnp.float32),
            jax.ShapeDtypeStruct((_ROWS, _COLS), jnp.float32),
        ],
        compiler_params=pltpu.CompilerParams(vmem_limit_bytes=100 * 1024 * 1024),
    )(emb_targets, emb_queries)
    return (out_t, out_q)


# speedup vs baseline: 1.0299x; 1.0034x over previous
import jax
import jax.numpy as jnp
from jax.experimental import pallas as pl
from jax.experimental.pallas import tpu as pltpu

_ROWS = 16384
_COLS = 256
_BLOCK_ROWS = 7680


def _copy_body(t_ref, q_ref, t_out, q_out):
    t_out[...] = t_ref[...]
    q_out[...] = q_ref[...]


def kernel(emb_targets, emb_queries):
    grid = (-(-_ROWS // _BLOCK_ROWS),)
    spec = pl.BlockSpec((_BLOCK_ROWS, _COLS), lambda i: (i, 0))
    out_t, out_q = pl.pallas_call(
        _copy_body,
        grid=grid,
        in_specs=[spec, spec],
        out_specs=[spec, spec],
        out_shape=[
            jax.ShapeDtypeStruct((_ROWS, _COLS), jnp.float32),
            jax.ShapeDtypeStruct((_ROWS, _COLS), jnp.float32),
        ],
        compiler_params=pltpu.CompilerParams(vmem_limit_bytes=100 * 1024 * 1024, dimension_semantics=("parallel",)),
    )(emb_targets, emb_queries)
    return (out_t, out_q)
